# baseline (speedup 1.0000x reference)
import functools

import jax
import jax.numpy as jnp
from jax import lax
from jax.experimental import pallas as pl
from jax.experimental.pallas import tpu as pltpu
from jax.experimental.pallas import tpu_sc as plsc

D = 256
DW = D // 2
E_TOT = 160000
C = 80
L = 16
NB = 4
T0 = 960
CNT0_BASE, CNT0_EXTRA = 60, 0
CNT1_BASE, CNT1_EXTRA = 64, 4
MAXCNT = 68


@functools.partial(
    pl.kernel,
    mesh=plsc.VectorSubcoreMesh(core_axis_name="c", subcore_axis_name="s"),
    out_type=jax.ShapeDtypeStruct((E_TOT,), jnp.float32),
    compiler_params=pltpu.CompilerParams(use_tc_tiling_on_sc=False),
    scratch_types=[
        pltpu.VMEM((MAXCNT * C,), jnp.int32),
        pltpu.VMEM((MAXCNT * C,), jnp.int32),
        pltpu.VMEM((NB, C, DW), jnp.int32),
        pltpu.VMEM((NB, C, DW), jnp.int32),
        pltpu.VMEM((NB, C), jnp.float32),
        pltpu.VMEM((L, 2 * L), jnp.float32),
        pltpu.SemaphoreType.DMA,
        pltpu.SemaphoreType.DMA,
        pltpu.SemaphoreType.DMA,
        pltpu.SemaphoreType.DMA,
        pltpu.SemaphoreType.DMA,
        pltpu.SemaphoreType.DMA,
        pltpu.SemaphoreType.DMA,
        pltpu.SemaphoreType.DMA,
        pltpu.SemaphoreType.DMA,
        pltpu.SemaphoreType.DMA,
        pltpu.SemaphoreType.DMA,
        pltpu.SemaphoreType.DMA,
    ],
)
def _score_kernel(xsrc, xdst, sidx_hbm, didx_hbm, out_hbm,
                  sidx_v, didx_v, s_rows, d_rows, out_v, red_v,
                  ss0, ss1, ss2, ss3, ds0, ds1, ds2, ds3,
                  os0, os1, os2, os3):
    cid = lax.axis_index("c")
    sid = lax.axis_index("s")
    ssems = (ss0, ss1, ss2, ss3)
    dsems = (ds0, ds1, ds2, ds3)
    osems = (os0, os1, os2, os3)

    n_chunks = jnp.where(
        cid == 0,
        CNT0_BASE + NB * (sid < CNT0_EXTRA).astype(jnp.int32),
        CNT1_BASE + NB * (sid < CNT1_EXTRA).astype(jnp.int32))
    chunk0 = jnp.where(
        cid == 0,
        CNT0_BASE * sid + NB * jnp.minimum(sid, CNT0_EXTRA),
        T0 + CNT1_BASE * sid + NB * jnp.minimum(sid, CNT1_EXTRA))
    base_e = chunk0 * C

    lane = lax.iota(jnp.int32, L)
    zeros = jnp.zeros((L,), jnp.float32)
    for j in range(L):
        red_v[j, pl.ds(L, L)] = zeros

    def copy_idx(n):
        pltpu.sync_copy(sidx_hbm.at[pl.ds(base_e, n * C)],
                        sidx_v.at[pl.ds(0, n * C)])
        pltpu.sync_copy(didx_hbm.at[pl.ds(base_e, n * C)],
                        didx_v.at[pl.ds(0, n * C)])

    @pl.when(jnp.logical_and(cid == 0, sid < CNT0_EXTRA))
    def _():
        copy_idx(CNT0_BASE + NB)

    @pl.when(jnp.logical_and(cid == 0, sid >= CNT0_EXTRA))
    def _():
        copy_idx(CNT0_BASE)

    @pl.when(jnp.logical_and(cid == 1, sid < CNT1_EXTRA))
    def _():
        copy_idx(CNT1_BASE + NB)

    @pl.when(jnp.logical_and(cid == 1, sid >= CNT1_EXTRA))
    def _():
        copy_idx(CNT1_BASE)

    def issue(c, b):
        pltpu.async_copy(xsrc.at[sidx_v.at[pl.ds(c * C, C)]], s_rows.at[b],
                         ssems[b])
        pltpu.async_copy(xdst.at[didx_v.at[pl.ds(c * C, C)]], d_rows.at[b],
                         dsems[b])

    def wait_gathers(c, b):
        pltpu.make_async_copy(xsrc.at[sidx_v.at[pl.ds(c * C, C)]],
                              s_rows.at[b], ssems[b]).wait()
        pltpu.make_async_copy(xdst.at[didx_v.at[pl.ds(c * C, C)]],
                              d_rows.at[b], dsems[b]).wait()

    def wait_out(b):
        pltpu.make_async_copy(out_v.at[b], out_hbm.at[pl.ds(base_e, C)],
                              osems[b]).wait()

    mask_pos = [jnp.logical_and(lane >= 4 * i, lane < 4 * i + 4)
                for i in range(4)]
    mask_mod = [(lane & 3) == q for q in range(4)]

    def compute(b):
        def grp_body(g, carry):
            qvs = [None] * 4
            for j in range(L):
                e = g * L + j
                i, q = j // 4, j % 4
                accs = [jnp.zeros((L,), jnp.float32) for _ in range(4)]
                for k in range(DW // L):
                    sv = s_rows[b, e, pl.ds(k * L, L)]
                    dv = d_rows[b, e, pl.ds(k * L, L)]
                    s_even = lax.bitcast_convert_type(sv << 16, jnp.float32)
                    d_even = lax.bitcast_convert_type(dv << 16, jnp.float32)
                    s_odd = lax.bitcast_convert_type(
                        sv & jnp.int32(-65536), jnp.float32)
                    d_odd = lax.bitcast_convert_type(
                        dv & jnp.int32(-65536), jnp.float32)
                    accs[(2 * k) % 4] = accs[(2 * k) % 4] + s_even * d_even
                    accs[(2 * k + 1) % 4] = (accs[(2 * k + 1) % 4]
                                             + s_odd * d_odd)
                t = (accs[0] + accs[1]) + (accs[2] + accs[3])
                for h in (8, 4):
                    red_v[j, pl.ds(0, L)] = t
                    t = t + red_v[j, pl.ds(h, L)]
                if i == 0:
                    qvs[q] = t
                else:
                    red_v[j, pl.ds(4 * i, L)] = t
                    shifted = red_v[j, pl.ds(0, L)]
                    qvs[q] = jnp.where(mask_pos[i], shifted, qvs[q])
            res = None
            for q in range(4):
                v = qvs[q]
                for h in (2, 1):
                    red_v[q, pl.ds(0, L)] = v
                    v = v + red_v[q, pl.ds(h, L)]
                if q == 0:
                    res = v
                else:
                    red_v[q, pl.ds(q, L)] = v
                    v = red_v[q, pl.ds(0, L)]
                    res = jnp.where(mask_mod[q], v, res)
            out_v[b, pl.ds(g * L, L)] = res
            return carry

        lax.fori_loop(0, C // L, grp_body, 0)

    issue(0, 0)
    issue(1, 1)
    issue(2, 2)

    def quad_body(i, carry):
        for b in range(NB):
            c = NB * i + b
            wait_gathers(c, b)
            nc = c + 3

            @pl.when(nc < n_chunks)
            def _():
                issue(nc, (b + 3) % NB)

            @pl.when(c >= NB)
            def _():
                wait_out(b)

            compute(b)
            pltpu.async_copy(out_v.at[b],
                             out_hbm.at[pl.ds(base_e + c * C, C)], osems[b])
        return carry

    lax.fori_loop(0, n_chunks // NB, quad_body, 0)
    for b in range(NB):
        wait_out(b)


def kernel(x_src, x_dst, src_idx, dst_idx):
    E = src_idx.shape[0]
    si = src_idx.astype(jnp.int32)
    di = dst_idx.astype(jnp.int32)
    xs_p = lax.bitcast_convert_type(
        x_src.astype(jnp.bfloat16).reshape(-1, DW, 2), jnp.int32)
    xd_p = lax.bitcast_convert_type(
        x_dst.astype(jnp.bfloat16).reshape(-1, DW, 2), jnp.int32)
    out = _score_kernel(xs_p, xd_p, si, di)
    return out.reshape(E, 1)


# speedup vs baseline: 1.0189x; 1.0189x over previous
import functools

import jax
import jax.numpy as jnp
from jax import lax
from jax.experimental import pallas as pl
from jax.experimental.pallas import tpu as pltpu
from jax.experimental.pallas import tpu_sc as plsc

D = 256
DW = D // 2
E_TOT = 160000
C = 80
L = 16
NB = 4
T0 = 988
CNT0_BASE, CNT0_EXTRA = 60, 7
CNT1_BASE, CNT1_EXTRA = 60, 13
MAXCNT = 64


@functools.partial(
    pl.kernel,
    mesh=plsc.VectorSubcoreMesh(core_axis_name="c", subcore_axis_name="s"),
    out_type=jax.ShapeDtypeStruct((E_TOT,), jnp.float32),
    compiler_params=pltpu.CompilerParams(use_tc_tiling_on_sc=False),
    scratch_types=[
        pltpu.VMEM((MAXCNT * C,), jnp.int32),
        pltpu.VMEM((MAXCNT * C,), jnp.int32),
        pltpu.VMEM((NB, C, DW), jnp.int32),
        pltpu.VMEM((NB, C, DW), jnp.int32),
        pltpu.VMEM((NB, C), jnp.float32),
        pltpu.VMEM((L, 2 * L), jnp.float32),
        pltpu.SemaphoreType.DMA,
        pltpu.SemaphoreType.DMA,
        pltpu.SemaphoreType.DMA,
        pltpu.SemaphoreType.DMA,
        pltpu.SemaphoreType.DMA,
        pltpu.SemaphoreType.DMA,
        pltpu.SemaphoreType.DMA,
        pltpu.SemaphoreType.DMA,
        pltpu.SemaphoreType.DMA,
        pltpu.SemaphoreType.DMA,
        pltpu.SemaphoreType.DMA,
        pltpu.SemaphoreType.DMA,
    ],
)
def _score_kernel(xsrc, xdst, sidx_hbm, didx_hbm, out_hbm,
                  sidx_v, didx_v, s_rows, d_rows, out_v, red_v,
                  ss0, ss1, ss2, ss3, ds0, ds1, ds2, ds3,
                  os0, os1, os2, os3):
    cid = lax.axis_index("c")
    sid = lax.axis_index("s")
    ssems = (ss0, ss1, ss2, ss3)
    dsems = (ds0, ds1, ds2, ds3)
    osems = (os0, os1, os2, os3)

    n_chunks = jnp.where(
        cid == 0,
        CNT0_BASE + NB * (sid < CNT0_EXTRA).astype(jnp.int32),
        CNT1_BASE + NB * (sid < CNT1_EXTRA).astype(jnp.int32))
    chunk0 = jnp.where(
        cid == 0,
        CNT0_BASE * sid + NB * jnp.minimum(sid, CNT0_EXTRA),
        T0 + CNT1_BASE * sid + NB * jnp.minimum(sid, CNT1_EXTRA))
    base_e = chunk0 * C

    lane = lax.iota(jnp.int32, L)
    zeros = jnp.zeros((L,), jnp.float32)
    for j in range(L):
        red_v[j, pl.ds(L, L)] = zeros

    def copy_idx(n):
        pltpu.sync_copy(sidx_hbm.at[pl.ds(base_e, n * C)],
                        sidx_v.at[pl.ds(0, n * C)])
        pltpu.sync_copy(didx_hbm.at[pl.ds(base_e, n * C)],
                        didx_v.at[pl.ds(0, n * C)])

    @pl.when(jnp.logical_and(cid == 0, sid < CNT0_EXTRA))
    def _():
        copy_idx(CNT0_BASE + NB)

    @pl.when(jnp.logical_and(cid == 0, sid >= CNT0_EXTRA))
    def _():
        copy_idx(CNT0_BASE)

    @pl.when(jnp.logical_and(cid == 1, sid < CNT1_EXTRA))
    def _():
        copy_idx(CNT1_BASE + NB)

    @pl.when(jnp.logical_and(cid == 1, sid >= CNT1_EXTRA))
    def _():
        copy_idx(CNT1_BASE)

    def issue(c, b):
        pltpu.async_copy(xsrc.at[sidx_v.at[pl.ds(c * C, C)]], s_rows.at[b],
                         ssems[b])
        pltpu.async_copy(xdst.at[didx_v.at[pl.ds(c * C, C)]], d_rows.at[b],
                         dsems[b])

    def wait_gathers(c, b):
        pltpu.make_async_copy(xsrc.at[sidx_v.at[pl.ds(c * C, C)]],
                              s_rows.at[b], ssems[b]).wait()
        pltpu.make_async_copy(xdst.at[didx_v.at[pl.ds(c * C, C)]],
                              d_rows.at[b], dsems[b]).wait()

    def wait_out(b):
        pltpu.make_async_copy(out_v.at[b], out_hbm.at[pl.ds(base_e, C)],
                              osems[b]).wait()

    mask_pos = [jnp.logical_and(lane >= 4 * i, lane < 4 * i + 4)
                for i in range(4)]
    mask_mod = [(lane & 3) == q for q in range(4)]

    def compute(b):
        def grp_body(g, carry):
            qvs = [None] * 4
            for j in range(L):
                e = g * L + j
                i, q = j // 4, j % 4
                accs = [jnp.zeros((L,), jnp.float32) for _ in range(4)]
                for k in range(DW // L):
                    sv = s_rows[b, e, pl.ds(k * L, L)]
                    dv = d_rows[b, e, pl.ds(k * L, L)]
                    s_even = lax.bitcast_convert_type(sv << 16, jnp.float32)
                    d_even = lax.bitcast_convert_type(dv << 16, jnp.float32)
                    s_odd = lax.bitcast_convert_type(
                        sv & jnp.int32(-65536), jnp.float32)
                    d_odd = lax.bitcast_convert_type(
                        dv & jnp.int32(-65536), jnp.float32)
                    accs[(2 * k) % 4] = accs[(2 * k) % 4] + s_even * d_even
                    accs[(2 * k + 1) % 4] = (accs[(2 * k + 1) % 4]
                                             + s_odd * d_odd)
                t = (accs[0] + accs[1]) + (accs[2] + accs[3])
                for h in (8, 4):
                    red_v[j, pl.ds(0, L)] = t
                    t = t + red_v[j, pl.ds(h, L)]
                if i == 0:
                    qvs[q] = t
                else:
                    red_v[j, pl.ds(4 * i, L)] = t
                    shifted = red_v[j, pl.ds(0, L)]
                    qvs[q] = jnp.where(mask_pos[i], shifted, qvs[q])
            res = None
            for q in range(4):
                v = qvs[q]
                for h in (2, 1):
                    red_v[q, pl.ds(0, L)] = v
                    v = v + red_v[q, pl.ds(h, L)]
                if q == 0:
                    res = v
                else:
                    red_v[q, pl.ds(q, L)] = v
                    v = red_v[q, pl.ds(0, L)]
                    res = jnp.where(mask_mod[q], v, res)
            out_v[b, pl.ds(g * L, L)] = res
            return carry

        lax.fori_loop(0, C // L, grp_body, 0)

    issue(0, 0)
    issue(1, 1)
    issue(2, 2)

    def quad_body(i, carry):
        for b in range(NB):
            c = NB * i + b
            wait_gathers(c, b)
            nc = c + 3

            @pl.when(nc < n_chunks)
            def _():
                issue(nc, (b + 3) % NB)

            @pl.when(c >= NB)
            def _():
                wait_out(b)

            compute(b)
            pltpu.async_copy(out_v.at[b],
                             out_hbm.at[pl.ds(base_e + c * C, C)], osems[b])
        return carry

    lax.fori_loop(0, n_chunks // NB, quad_body, 0)
    for b in range(NB):
        wait_out(b)


def kernel(x_src, x_dst, src_idx, dst_idx):
    E = src_idx.shape[0]
    si = src_idx.astype(jnp.int32)
    di = dst_idx.astype(jnp.int32)
    xs_p = lax.bitcast_convert_type(
        x_src.astype(jnp.bfloat16).reshape(-1, DW, 2), jnp.int32)
    xd_p = lax.bitcast_convert_type(
        x_dst.astype(jnp.bfloat16).reshape(-1, DW, 2), jnp.int32)
    out = _score_kernel(xs_p, xd_p, si, di)
    return out.reshape(E, 1)
